# baseline (speedup 1.0000x reference)
import functools

import jax
import jax.numpy as jnp
from jax import lax
from jax.experimental import pallas as pl
from jax.experimental.pallas import tpu as pltpu
from jax.experimental.pallas import tpu_sc as plsc

NC = 2
NS = 16
NW = NC * NS
L = 16
BLK = 128


def _make_kernel(B, F, V, D):
    assert D == L
    assert B % (NW * BLK) == 0
    nblk = B // (NW * BLK)
    rows_per_blk = F * BLK

    mesh = plsc.VectorSubcoreMesh(core_axis_name="c", subcore_axis_name="s")

    @functools.partial(
        pl.kernel,
        out_type=jax.ShapeDtypeStruct((B,), jnp.float32),
        mesh=mesh,
        compiler_params=pltpu.CompilerParams(
            needs_layout_passes=False, use_tc_tiling_on_sc=False),
        scratch_types=[
            pltpu.VMEM((2, F, BLK), jnp.int32),
            pltpu.VMEM((2, rows_per_blk, L), jnp.float32),
            pltpu.VMEM((F, L), jnp.float32),
            pltpu.VMEM((L, L), jnp.float32),
            pltpu.VMEM((BLK,), jnp.float32),
            pltpu.SemaphoreType.DMA,
            pltpu.SemaphoreType.DMA,
        ],
    )
    def kern(tab_hbm, idx_hbm, w_hbm, out_hbm, idx_v, rows_v, w_v, colbuf,
             out_blk, sem0, sem1):
        sems = (sem0, sem1)
        wid = lax.axis_index("s") * NC + lax.axis_index("c")

        pltpu.sync_copy(w_hbm, w_v)
        wv = [w_v[f, :] for f in range(F)]
        iota16 = lax.iota(jnp.int32, L)

        def fire(slot, blk):
            g = wid * nblk + blk
            pltpu.sync_copy(idx_hbm.at[g], idx_v.at[slot])

            @pl.loop(0, F)
            def _(f):
                pltpu.async_copy(
                    tab_hbm.at[idx_v.at[slot, f]],
                    rows_v.at[slot, pl.ds(f * BLK, BLK)],
                    sems[slot],
                )

        def drain(slot):
            pltpu.make_async_copy(
                tab_hbm.at[pl.ds(0, rows_per_blk)], rows_v.at[slot], sems[slot]
            ).wait()

        def compute(slot, blk):
            @pl.loop(0, BLK // L)
            def _(grp):
                for l in range(L):
                    b = grp * L + l
                    acc = rows_v[slot, b * F, :] * wv[0]
                    for f in range(1, F):
                        acc = acc + rows_v[slot, b * F + f, :] * wv[f]
                    colbuf[l, :] = acc
                tot = plsc.load_gather(colbuf, [iota16, jnp.zeros((L,), jnp.int32)])
                for d in range(1, L):
                    tot = tot + plsc.load_gather(
                        colbuf, [iota16, jnp.full((L,), d, jnp.int32)])
                out_blk[pl.ds(grp * L, L)] = 1.0 / (1.0 + jnp.exp(-tot))

            base = wid * (nblk * BLK) + blk * BLK
            pltpu.sync_copy(out_blk, out_hbm.at[pl.ds(base, BLK)])

        fire(0, 0)
        for blk in range(nblk):
            slot = blk % 2
            if blk + 1 < nblk:
                fire(1 - slot, blk + 1)
            drain(slot)
            compute(slot, blk)

    return kern


def kernel(indices, tables, W):
    B, F = indices.shape
    _, V, D = tables.shape
    tab2 = tables.reshape(F * V, D)
    flat = indices.astype(jnp.int32) + (jnp.arange(F, dtype=jnp.int32) * V)[None, :]
    idx3 = flat.reshape(B // BLK, F, BLK)
    w2 = W.reshape(F, D).astype(jnp.float32)
    out = _make_kernel(B, F, V, D)(tab2, idx3, w2)
    return out.reshape(B, 1)


# speedup vs baseline: 3.6594x; 3.6594x over previous
import functools

import jax
import jax.numpy as jnp
from jax import lax
from jax.experimental import pallas as pl
from jax.experimental.pallas import tpu as pltpu
from jax.experimental.pallas import tpu_sc as plsc

NC = 2
NS = 16
NW = NC * NS
L = 16
ICHUNK = 128


def _make_tc_project(F, D, V):
    def body(t_ref, w_ref, o_ref):
        f = pl.program_id(0)
        t = t_ref[0]
        acc = t[0:1, :] * w_ref[0, f]
        for d in range(1, D):
            acc = acc + t[d:d + 1, :] * w_ref[d, f]
        o_ref[0] = acc

    return pl.pallas_call(
        body,
        grid=(F,),
        in_specs=[
            pl.BlockSpec((1, D, V), lambda f: (f, 0, 0)),
            pl.BlockSpec(memory_space=pltpu.SMEM),
        ],
        out_specs=pl.BlockSpec((1, 1, V), lambda f: (f, 0, 0)),
        out_shape=jax.ShapeDtypeStruct((F, 1, V), jnp.float32),
    )


def _make_sc_gather(B, F, V):
    bpw = B // NW
    npos = bpw * F
    nchunk = npos // ICHUNK

    mesh = plsc.VectorSubcoreMesh(core_axis_name="c", subcore_axis_name="s")

    @functools.partial(
        pl.kernel,
        out_type=jax.ShapeDtypeStruct((B,), jnp.float32),
        mesh=mesh,
        compiler_params=pltpu.CompilerParams(
            needs_layout_passes=False, use_tc_tiling_on_sc=False),
        scratch_types=[
            pltpu.VMEM((nchunk, ICHUNK), jnp.int32),
            pltpu.VMEM((npos,), jnp.float32),
            pltpu.VMEM((bpw,), jnp.float32),
            pltpu.SemaphoreType.DMA,
        ],
    )
    def kern(scores_hbm, idx_hbm, out_hbm, idx_v, dst_v, out_w, sem):
        wid = lax.axis_index("s") * NC + lax.axis_index("c")
        pltpu.sync_copy(idx_hbm.at[wid], idx_v)

        @pl.loop(0, nchunk)
        def _(c):
            pltpu.async_copy(
                scores_hbm.at[idx_v.at[c]],
                dst_v.at[pl.ds(c * ICHUNK, ICHUNK)],
                sem,
            )

        pltpu.make_async_copy(
            scores_hbm.at[pl.ds(0, npos)], dst_v, sem
        ).wait()

        @pl.loop(0, bpw // L)
        def _(g):
            tot = dst_v[pl.ds(g * L, L)]
            for f in range(1, F):
                tot = tot + dst_v[pl.ds(f * bpw + g * L, L)]
            out_w[pl.ds(g * L, L)] = 1.0 / (1.0 + jnp.exp(-tot))

        pltpu.sync_copy(out_w, out_hbm.at[pl.ds(wid * bpw, bpw)])

    return kern


def kernel(indices, tables, W):
    B, F = indices.shape
    _, V, D = tables.shape
    bpw = B // NW
    qpw = bpw // ICHUNK

    w2t = W.reshape(F, D).astype(jnp.float32).T
    scores = _make_tc_project(F, D, V)(tables.transpose(0, 2, 1), w2t)

    flat_t = indices.astype(jnp.int32).T + (jnp.arange(F, dtype=jnp.int32) * V)[:, None]
    idx4 = (
        flat_t.reshape(F, NW, qpw, ICHUNK)
        .transpose(1, 0, 2, 3)
        .reshape(NW, F * qpw, ICHUNK)
    )
    out = _make_sc_gather(B, F, V)(scores.reshape(F * V), idx4)
    return out.reshape(B, 1)


# speedup vs baseline: 10.8730x; 2.9712x over previous
import functools

import jax
import jax.numpy as jnp
from jax import lax
from jax.experimental import pallas as pl
from jax.experimental.pallas import tpu as pltpu
from jax.experimental.pallas import tpu_sc as plsc

NC = 2
NS = 16
NW = NC * NS
L = 16
ICHUNK = 128


def _make_tc_project(F, D, V, VP):
    def body(t_ref, w_ref, o_ref):
        f = pl.program_id(0)
        t = t_ref[0]
        ids = lax.broadcasted_iota(jnp.int32, (D, 1), 0)
        wcol = jnp.full((D, 1), w_ref[0, f], jnp.float32)
        for d in range(1, D):
            wcol = jnp.where(ids == d, w_ref[d, f], wcol)
        score = jnp.sum(t * wcol, axis=0)
        o_ref[pl.ds(0, V)] = score

    return pl.pallas_call(
        body,
        grid=(F,),
        in_specs=[
            pl.BlockSpec((1, D, V), lambda f: (f, 0, 0)),
            pl.BlockSpec(memory_space=pltpu.SMEM),
        ],
        out_specs=pl.BlockSpec((VP,), lambda f: (f,)),
        out_shape=jax.ShapeDtypeStruct((F * VP,), jnp.float32),
    )


def _make_sc_gather(B, F, V):
    bpw = B // NW
    npos = bpw * F
    nchunk = npos // ICHUNK

    mesh = plsc.VectorSubcoreMesh(core_axis_name="c", subcore_axis_name="s")

    @functools.partial(
        pl.kernel,
        out_type=jax.ShapeDtypeStruct((B,), jnp.float32),
        mesh=mesh,
        compiler_params=pltpu.CompilerParams(
            needs_layout_passes=False, use_tc_tiling_on_sc=False),
        scratch_types=[
            pltpu.VMEM((nchunk, ICHUNK), jnp.int32),
            pltpu.VMEM((npos,), jnp.float32),
            pltpu.VMEM((bpw,), jnp.float32),
            pltpu.SemaphoreType.DMA,
        ],
    )
    def kern(scores_hbm, idx_hbm, out_hbm, idx_v, dst_v, out_w, sem):
        wid = lax.axis_index("s") * NC + lax.axis_index("c")
        pltpu.sync_copy(idx_hbm.at[wid], idx_v)

        @pl.loop(0, nchunk)
        def _(c):
            pltpu.async_copy(
                scores_hbm.at[idx_v.at[c]],
                dst_v.at[pl.ds(c * ICHUNK, ICHUNK)],
                sem,
            )

        pltpu.make_async_copy(
            scores_hbm.at[pl.ds(0, npos)], dst_v, sem
        ).wait()

        @pl.loop(0, bpw // L)
        def _(g):
            tot = dst_v[pl.ds(g * L, L)]
            for f in range(1, F):
                tot = tot + dst_v[pl.ds(f * bpw + g * L, L)]
            out_w[pl.ds(g * L, L)] = 1.0 / (1.0 + jnp.exp(-tot))

        pltpu.sync_copy(out_w, out_hbm.at[pl.ds(wid * bpw, bpw)])

    return kern


def kernel(indices, tables, W):
    B, F = indices.shape
    _, V, D = tables.shape
    bpw = B // NW
    qpw = bpw // ICHUNK

    VP = (V + 1023) // 1024 * 1024
    w2t = W.reshape(F, D).astype(jnp.float32).T
    scores = _make_tc_project(F, D, V, VP)(tables.transpose(0, 2, 1), w2t)

    flat_t = indices.astype(jnp.int32).T + (jnp.arange(F, dtype=jnp.int32) * VP)[:, None]
    idx4 = (
        flat_t.reshape(F, NW, qpw, ICHUNK)
        .transpose(1, 0, 2, 3)
        .reshape(NW, F * qpw, ICHUNK)
    )
    out = _make_sc_gather(B, F, V)(scores, idx4)
    return out.reshape(B, 1)


# speedup vs baseline: 11.0247x; 1.0140x over previous
import functools

import jax
import jax.numpy as jnp
from jax import lax
from jax.experimental import pallas as pl
from jax.experimental.pallas import tpu as pltpu
from jax.experimental.pallas import tpu_sc as plsc

NC = 2
NS = 16
NW = NC * NS
L = 16
ICHUNK = 128


def _make_tc_project(F, D, V, VP, f0, FH):
    def body(t_ref, w_ref, o_ref):
        f = pl.program_id(0) + f0
        t = t_ref[0]
        ids = lax.broadcasted_iota(jnp.int32, (D, 1), 0)
        wcol = jnp.full((D, 1), w_ref[0, f], jnp.float32)
        for d in range(1, D):
            wcol = jnp.where(ids == d, w_ref[d, f], wcol)
        score = jnp.sum(t * wcol, axis=0)
        o_ref[pl.ds(0, V)] = score

    return pl.pallas_call(
        body,
        grid=(FH,),
        in_specs=[
            pl.BlockSpec((1, D, V), lambda f: (f + f0, 0, 0)),
            pl.BlockSpec(memory_space=pltpu.SMEM),
        ],
        out_specs=pl.BlockSpec((VP,), lambda f: (f,)),
        out_shape=jax.ShapeDtypeStruct((FH * VP,), jnp.float32),
    )


def _make_sc_gather(B, F, f0, FH, qpw, last):
    bpw = B // NW
    npos = bpw * FH

    mesh = plsc.VectorSubcoreMesh(core_axis_name="c", subcore_axis_name="s")

    scratch = [
        pltpu.VMEM((FH, qpw, ICHUNK), jnp.int32),
        pltpu.VMEM((npos,), jnp.float32),
        pltpu.VMEM((bpw,), jnp.float32),
        pltpu.SemaphoreType.DMA,
    ]
    if last:
        scratch.append(pltpu.VMEM((bpw,), jnp.float32))

    @functools.partial(
        pl.kernel,
        out_type=jax.ShapeDtypeStruct((B,), jnp.float32),
        mesh=mesh,
        compiler_params=pltpu.CompilerParams(
            needs_layout_passes=False, use_tc_tiling_on_sc=False),
        scratch_types=scratch,
    )
    def kern(scores_hbm, idx_hbm, *rest):
        if last:
            part_hbm, out_hbm, idx_v, dst_v, out_w, sem, part_v = rest
        else:
            out_hbm, idx_v, dst_v, out_w, sem = rest
        wid = lax.axis_index("s") * NC + lax.axis_index("c")
        pltpu.sync_copy(idx_hbm.at[pl.ds(f0, FH), wid], idx_v)
        if last:
            pltpu.sync_copy(part_hbm.at[pl.ds(wid * bpw, bpw)], part_v)

        @pl.loop(0, FH)
        def _(f):
            for q in range(qpw):
                pltpu.async_copy(
                    scores_hbm.at[idx_v.at[f, q]],
                    dst_v.at[pl.ds((f * qpw + q) * ICHUNK, ICHUNK)],
                    sem,
                )

        pltpu.make_async_copy(
            scores_hbm.at[pl.ds(0, npos)], dst_v, sem
        ).wait()

        @pl.loop(0, bpw // L)
        def _(g):
            tot = dst_v[pl.ds(g * L, L)]
            for f in range(1, FH):
                tot = tot + dst_v[pl.ds(f * bpw + g * L, L)]
            if last:
                tot = tot + part_v[pl.ds(g * L, L)]
                tot = 1.0 / (1.0 + jnp.exp(-tot))
            out_w[pl.ds(g * L, L)] = tot

        pltpu.sync_copy(out_w, out_hbm.at[pl.ds(wid * bpw, bpw)])

    return kern


def kernel(indices, tables, W):
    B, F = indices.shape
    _, V, D = tables.shape
    bpw = B // NW
    qpw = bpw // ICHUNK
    VP = (V + 1023) // 1024 * 1024
    FH = F // 2

    w2t = W.reshape(F, D).astype(jnp.float32).T
    tables_t = tables.transpose(0, 2, 1)
    scores0 = _make_tc_project(F, D, V, VP, 0, FH)(tables_t, w2t)
    scores1 = _make_tc_project(F, D, V, VP, FH, F - FH)(tables_t, w2t)

    off = (jnp.arange(F, dtype=jnp.int32) % FH) * VP
    flat_t = indices.astype(jnp.int32).T + off[:, None]
    idx4 = flat_t.reshape(F, NW, qpw, ICHUNK)

    part = _make_sc_gather(B, F, 0, FH, qpw, False)(scores0, idx4)
    out = _make_sc_gather(B, F, FH, F - FH, qpw, True)(scores1, idx4, part)
    return out.reshape(B, 1)
